# baseline (speedup 1.0000x reference)
import functools

import jax
import jax.numpy as jnp
from jax import lax
from jax.experimental import pallas as pl
from jax.experimental.pallas import tpu as pltpu
from jax.experimental.pallas import tpu_sc as plsc

N_TP = 4
N_GP = 2
E = 8
RAWOB = 512
NA = 128
DH = 512
T = 128
A = 16
N = T * A
DEXT = 16
DPAD = 128
M = 128
NB = N // M

NC = 2
NS = 16
NW = NC * NS
TOK_W = N // NW
L = 16


def _route_body(pick_hbm, ht_hbm, gp_hbm, obs_hbm,
                xo_hbm, xe_hbm, dest_hbm, offs_hbm,
                pick_all, ht_v, gp_v, dest_v, xo_buf, xe_buf,
                offs_v, sem, sem2):
    wid = lax.axis_index("s") * NC + lax.axis_index("c")
    base_tok = wid * TOK_W
    pltpu.sync_copy(pick_hbm, pick_all)
    pltpu.sync_copy(ht_hbm.at[pl.ds(base_tok, TOK_W)], ht_v)
    pltpu.sync_copy(gp_hbm.at[pl.ds(wid * L, L)], gp_v)
    obs_cp = pltpu.async_copy(obs_hbm.at[pl.ds(base_tok, TOK_W)], xo_buf, sem2)

    lane = lax.iota(jnp.int32, L)
    myvreg0 = wid * (TOK_W // L)

    def hist_body(j, carry):
        total, pre = carry
        v = pick_all[pl.ds(j * L, L)]
        oh = jnp.zeros((L,), jnp.int32)
        for e in range(E):
            c = plsc.all_reduce_population_count(v == e)
            oh = oh + jnp.where(lane == e, c, 0)
        return total + oh, pre + jnp.where(j < myvreg0, oh, 0)

    total, pre = lax.fori_loop(
        0, N // L, hist_body,
        (jnp.zeros((L,), jnp.int32), jnp.zeros((L,), jnp.int32)))
    offs_excl = plsc.cumsum(total) - total

    @pl.when(wid == 0)
    def _():
        offs_v[...] = offs_excl
        pltpu.sync_copy(offs_v, offs_hbm)

    base = offs_excl + pre
    for v in range(TOK_W // L):
        pv = pick_all[pl.ds(base_tok + v * L, L)]
        dv = jnp.zeros((L,), jnp.int32)
        vcnt = jnp.zeros((L,), jnp.int32)
        for e in range(E):
            m = pv == e
            cs = plsc.cumsum(m.astype(jnp.int32))
            base_e = jnp.sum(jnp.where(lane == e, base, 0))
            dv = jnp.where(m, base_e + cs - 1, dv)
            c = plsc.all_reduce_population_count(m)
            vcnt = vcnt + jnp.where(lane == e, c, 0)
        dest_v[pl.ds(v * L, L)] = dv
        base = base + vcnt

    zero16 = jnp.zeros((L,), jnp.float32)
    for r in range(TOK_W):
        xe_buf[r, pl.ds(0, L)] = zero16
    gpr = gp_v[...]
    for v in range(TOK_W // L):
        rvec = lane + v * L
        htv = ht_v[pl.ds(v * L, L)]
        plsc.store_scatter(xe_buf, [rvec, jnp.zeros((L,), jnp.int32)],
                           htv.astype(jnp.float32))
        for k in range(N_TP):
            gpk = jnp.sum(jnp.where(lane == v * N_TP + k, gpr, 0))
            val = jnp.where(htv == k, -1.0, gpk.astype(jnp.float32))
            plsc.store_scatter(xe_buf, [rvec, jnp.full((L,), 1 + k, jnp.int32)],
                               val)

    pltpu.sync_copy(dest_v, dest_hbm.at[pl.ds(base_tok, TOK_W)])
    obs_cp.wait()
    cp1 = pltpu.async_copy(xo_buf, xo_hbm.at[dest_v], sem)
    cp2 = pltpu.async_copy(xe_buf, xe_hbm.at[dest_v], sem2)
    cp1.wait()
    cp2.wait()


_route = functools.partial(
    pl.kernel,
    out_type=(
        jax.ShapeDtypeStruct((N, RAWOB), jnp.float32),
        jax.ShapeDtypeStruct((N, DPAD), jnp.float32),
        jax.ShapeDtypeStruct((N,), jnp.int32),
        jax.ShapeDtypeStruct((L,), jnp.int32),
    ),
    mesh=plsc.VectorSubcoreMesh(core_axis_name="c", subcore_axis_name="s", num_cores=NC, num_subcores=NS),
    compiler_params=pltpu.CompilerParams(needs_layout_passes=False),
    scratch_types=(
        pltpu.VMEM((N,), jnp.int32),
        pltpu.VMEM((TOK_W,), jnp.int32),
        pltpu.VMEM((L,), jnp.int32),
        pltpu.VMEM((TOK_W,), jnp.int32),
        pltpu.VMEM((TOK_W, RAWOB), jnp.float32),
        pltpu.VMEM((TOK_W, DPAD), jnp.float32),
        pltpu.VMEM((L,), jnp.int32),
        pltpu.SemaphoreType.DMA,
        pltpu.SemaphoreType.DMA,
    ),
)(_route_body)


def _gather_body(ls_hbm, vs_hbm, dest_hbm, lf_hbm, vf_hbm,
                 idx_v, lbuf, vbuf, sem):
    wid = lax.axis_index("s") * NC + lax.axis_index("c")
    base_tok = wid * TOK_W
    pltpu.sync_copy(dest_hbm.at[pl.ds(base_tok, TOK_W)], idx_v)
    pltpu.async_copy(ls_hbm.at[idx_v], lbuf, sem).wait()
    pltpu.async_copy(vs_hbm.at[idx_v], vbuf, sem).wait()
    pltpu.sync_copy(lbuf, lf_hbm.at[pl.ds(base_tok, TOK_W)])
    pltpu.sync_copy(vbuf, vf_hbm.at[pl.ds(base_tok, TOK_W)])


_gather = functools.partial(
    pl.kernel,
    out_type=(
        jax.ShapeDtypeStruct((N, NA), jnp.float32),
        jax.ShapeDtypeStruct((N, DPAD), jnp.float32),
    ),
    mesh=plsc.VectorSubcoreMesh(core_axis_name="c", subcore_axis_name="s", num_cores=NC, num_subcores=NS),
    compiler_params=pltpu.CompilerParams(needs_layout_passes=False),
    scratch_types=(
        pltpu.VMEM((TOK_W,), jnp.int32),
        pltpu.VMEM((TOK_W, NA), jnp.float32),
        pltpu.VMEM((TOK_W, DPAD), jnp.float32),
        pltpu.SemaphoreType.DMA,
    ),
)(_gather_body)


def _moe_body(offs_ref, xo_ref, xe_ref, W1_ref, W1e_ref, b1_ref, W2_ref,
              b2_ref, Wc1_ref, Wc1e_ref, bc1_ref, Wc2_ref, bc2_ref,
              lo_ref, vo_ref):
    i = pl.program_id(0)
    row0 = i * M
    xo = xo_ref[...].astype(jnp.bfloat16)
    xe = xe_ref[:, :DEXT].astype(jnp.bfloat16)
    hc = jnp.maximum(
        jnp.dot(xo, Wc1_ref[...], preferred_element_type=jnp.float32)
        + jnp.dot(xe, Wc1e_ref[...], preferred_element_type=jnp.float32)
        + bc1_ref[...], 0.0)
    vo_ref[...] = (jnp.dot(hc.astype(jnp.bfloat16), Wc2_ref[...],
                           preferred_element_type=jnp.float32)
                   + bc2_ref[...])
    rid = row0 + lax.broadcasted_iota(jnp.int32, (M, 1), 0)

    def body(e, carry):
        s = offs_ref[e]
        t = offs_ref[e + 1]

        @pl.when((s < row0 + M) & (t > row0))
        def _():
            h = jnp.maximum(
                jnp.dot(xo, W1_ref[e], preferred_element_type=jnp.float32)
                + jnp.dot(xe, W1e_ref[e], preferred_element_type=jnp.float32)
                + b1_ref[e], 0.0)
            lg = (jnp.dot(h.astype(jnp.bfloat16), W2_ref[e],
                          preferred_element_type=jnp.float32)
                  + b2_ref[e])
            msk = (rid >= s) & (rid < t)
            lo_ref[...] = jnp.where(msk, lg, lo_ref[...])
        return carry

    lax.fori_loop(0, E, body, 0)


def _moe(offs, xo_s, xe_s, W1, W1e, b1, W2, b2, Wc1, Wc1e, bc1, Wc2p, bc2p):
    grid_spec = pltpu.PrefetchScalarGridSpec(
        num_scalar_prefetch=1,
        grid=(NB,),
        in_specs=[
            pl.BlockSpec((M, RAWOB), lambda i, s: (i, 0)),
            pl.BlockSpec((M, DPAD), lambda i, s: (i, 0)),
            pl.BlockSpec((E, DH, DH), lambda i, s: (0, 0, 0)),
            pl.BlockSpec((E, DEXT, DH), lambda i, s: (0, 0, 0)),
            pl.BlockSpec((E, DH), lambda i, s: (0, 0)),
            pl.BlockSpec((E, DH, NA), lambda i, s: (0, 0, 0)),
            pl.BlockSpec((E, NA), lambda i, s: (0, 0)),
            pl.BlockSpec((DH, DH), lambda i, s: (0, 0)),
            pl.BlockSpec((DEXT, DH), lambda i, s: (0, 0)),
            pl.BlockSpec((1, DH), lambda i, s: (0, 0)),
            pl.BlockSpec((DH, DPAD), lambda i, s: (0, 0)),
            pl.BlockSpec((1, DPAD), lambda i, s: (0, 0)),
        ],
        out_specs=[
            pl.BlockSpec((M, NA), lambda i, s: (i, 0)),
            pl.BlockSpec((M, DPAD), lambda i, s: (i, 0)),
        ],
    )
    return pl.pallas_call(
        _moe_body,
        grid_spec=grid_spec,
        out_shape=(
            jax.ShapeDtypeStruct((N, NA), jnp.float32),
            jax.ShapeDtypeStruct((N, DPAD), jnp.float32),
        ),
        compiler_params=pltpu.CompilerParams(
            dimension_semantics=("arbitrary",),
            vmem_limit_bytes=100 * 1024 * 1024,
        ),
    )(offs, xo_s, xe_s, W1, W1e, b1, W2, b2, Wc1, Wc1e, bc1, Wc2p, bc2p)


def kernel(hete_pick, obs, hete_type, gp_sel_summary, thread_index,
           W1, b1, W2, b2, Wc1, bc1, Wc2, bc2):
    pick = hete_pick.reshape(N).astype(jnp.int32)
    ht = hete_type.reshape(N).astype(jnp.int32)
    gpf = gp_sel_summary.reshape(T * N_TP).astype(jnp.int32)
    obs2 = obs.reshape(N, RAWOB)
    W1e = jnp.pad(W1[:, RAWOB:, :], ((0, 0), (0, DEXT - 5), (0, 0)))
    Wc1e = jnp.pad(Wc1[RAWOB:, :], ((0, DEXT - 5), (0, 0)))
    Wc2p = jnp.pad(Wc2, ((0, 0), (0, DPAD - 1)))
    W1b = W1.astype(jnp.bfloat16)
    W1eb = W1e.astype(jnp.bfloat16)
    W2b = W2.astype(jnp.bfloat16)
    Wc1b = Wc1.astype(jnp.bfloat16)
    Wc1eb = Wc1e.astype(jnp.bfloat16)
    Wc2pb = Wc2p.astype(jnp.bfloat16)
    bc2p = jnp.pad(bc2, (0, DPAD - 1)).reshape(1, DPAD)
    bc1_2 = bc1.reshape(1, DH)

    xo_s, xe_s, dest, offs = _route(pick, ht, gpf, obs2)
    ls, vs = _moe(offs, xo_s, xe_s, W1b, W1eb, b1, W2b, b2,
                  Wc1b, Wc1eb, bc1_2, Wc2pb, bc2p)
    lf, vf = _gather(ls, vs, dest)
    return lf.reshape(T, A, NA), vf[:, :1].reshape(T, A, 1)


# speedup vs baseline: 1.0362x; 1.0362x over previous
import functools

import jax
import jax.numpy as jnp
from jax import lax
from jax.experimental import pallas as pl
from jax.experimental.pallas import tpu as pltpu
from jax.experimental.pallas import tpu_sc as plsc

N_TP = 4
N_GP = 2
E = 8
RAWOB = 512
NA = 128
DH = 512
T = 128
A = 16
N = T * A
DEXT = 16
DPAD = 128
VW = 8
M = 256
NB = N // M

NC = 2
NS = 16
NW = NC * NS
TOK_W = N // NW
L = 16


def _route_body(pick_hbm, ht_hbm, gp_hbm, obs_hbm,
                xo_hbm, xe_hbm, dest_hbm, offs_hbm,
                pick_all, ht_v, gp_v, dest_v, xo_buf, xe_buf,
                offs_v, sem, sem2):
    wid = lax.axis_index("s") * NC + lax.axis_index("c")
    base_tok = wid * TOK_W
    pltpu.sync_copy(pick_hbm, pick_all)
    pltpu.sync_copy(ht_hbm.at[pl.ds(base_tok, TOK_W)], ht_v)
    pltpu.sync_copy(gp_hbm.at[pl.ds(wid * L, L)], gp_v)
    obs_cp = pltpu.async_copy(obs_hbm.at[pl.ds(base_tok, TOK_W)], xo_buf, sem2)

    lane = lax.iota(jnp.int32, L)
    myvreg0 = wid * (TOK_W // L)

    def hist_body(j, carry):
        total, pre = carry
        v = pick_all[pl.ds(j * L, L)]
        oh = jnp.zeros((L,), jnp.int32)
        for e in range(E):
            c = plsc.all_reduce_population_count(v == e)
            oh = oh + jnp.where(lane == e, c, 0)
        return total + oh, pre + jnp.where(j < myvreg0, oh, 0)

    total, pre = lax.fori_loop(
        0, N // L, hist_body,
        (jnp.zeros((L,), jnp.int32), jnp.zeros((L,), jnp.int32)))
    offs_excl = plsc.cumsum(total) - total

    @pl.when(wid == 0)
    def _():
        offs_v[...] = offs_excl
        pltpu.sync_copy(offs_v, offs_hbm)

    base = offs_excl + pre
    for v in range(TOK_W // L):
        pv = pick_all[pl.ds(base_tok + v * L, L)]
        dv = jnp.zeros((L,), jnp.int32)
        vcnt = jnp.zeros((L,), jnp.int32)
        for e in range(E):
            m = pv == e
            cs = plsc.cumsum(m.astype(jnp.int32))
            base_e = jnp.sum(jnp.where(lane == e, base, 0))
            dv = jnp.where(m, base_e + cs - 1, dv)
            c = plsc.all_reduce_population_count(m)
            vcnt = vcnt + jnp.where(lane == e, c, 0)
        dest_v[pl.ds(v * L, L)] = dv
        base = base + vcnt

    zero16 = jnp.zeros((L,), jnp.float32)
    for r in range(TOK_W):
        xe_buf[r, pl.ds(0, L)] = zero16
    gpr = gp_v[...]
    for v in range(TOK_W // L):
        rvec = lane + v * L
        htv = ht_v[pl.ds(v * L, L)]
        plsc.store_scatter(xe_buf, [rvec, jnp.zeros((L,), jnp.int32)],
                           htv.astype(jnp.float32))
        for k in range(N_TP):
            gpk = jnp.sum(jnp.where(lane == v * N_TP + k, gpr, 0))
            val = jnp.where(htv == k, -1.0, gpk.astype(jnp.float32))
            plsc.store_scatter(xe_buf, [rvec, jnp.full((L,), 1 + k, jnp.int32)],
                               val)

    pltpu.sync_copy(dest_v, dest_hbm.at[pl.ds(base_tok, TOK_W)])
    obs_cp.wait()
    cp1 = pltpu.async_copy(xo_buf, xo_hbm.at[dest_v], sem)
    cp2 = pltpu.async_copy(xe_buf, xe_hbm.at[dest_v], sem2)
    cp1.wait()
    cp2.wait()


_route = functools.partial(
    pl.kernel,
    out_type=(
        jax.ShapeDtypeStruct((N, RAWOB), jnp.float32),
        jax.ShapeDtypeStruct((N, DPAD), jnp.float32),
        jax.ShapeDtypeStruct((N,), jnp.int32),
        jax.ShapeDtypeStruct((L,), jnp.int32),
    ),
    mesh=plsc.VectorSubcoreMesh(core_axis_name="c", subcore_axis_name="s", num_cores=NC, num_subcores=NS),
    compiler_params=pltpu.CompilerParams(needs_layout_passes=False),
    scratch_types=(
        pltpu.VMEM((N,), jnp.int32),
        pltpu.VMEM((TOK_W,), jnp.int32),
        pltpu.VMEM((L,), jnp.int32),
        pltpu.VMEM((TOK_W,), jnp.int32),
        pltpu.VMEM((TOK_W, RAWOB), jnp.float32),
        pltpu.VMEM((TOK_W, DPAD), jnp.float32),
        pltpu.VMEM((L,), jnp.int32),
        pltpu.SemaphoreType.DMA,
        pltpu.SemaphoreType.DMA,
    ),
)(_route_body)


def _gather_body(ls_hbm, dest_hbm, lf_hbm, idx_v, lbuf, sem):
    wid = lax.axis_index("s") * NC + lax.axis_index("c")
    base_tok = wid * TOK_W
    pltpu.sync_copy(dest_hbm.at[pl.ds(base_tok, TOK_W)], idx_v)
    pltpu.async_copy(ls_hbm.at[idx_v], lbuf, sem).wait()
    pltpu.sync_copy(lbuf, lf_hbm.at[pl.ds(base_tok, TOK_W)])


_gather = functools.partial(
    pl.kernel,
    out_type=jax.ShapeDtypeStruct((N, NA), jnp.float32),
    mesh=plsc.VectorSubcoreMesh(core_axis_name="c", subcore_axis_name="s", num_cores=NC, num_subcores=NS),
    compiler_params=pltpu.CompilerParams(needs_layout_passes=False),
    scratch_types=(
        pltpu.VMEM((TOK_W,), jnp.int32),
        pltpu.VMEM((TOK_W, NA), jnp.float32),
        pltpu.SemaphoreType.DMA,
    ),
)(_gather_body)


def _moe_body(offs_ref, xo_ref, xe_ref, W1_ref, W1e_ref, b1_ref, W2_ref,
              b2_ref, lo_ref):
    i = pl.program_id(0)
    row0 = i * M
    xo = xo_ref[...].astype(jnp.bfloat16)
    xe = xe_ref[:, :DEXT].astype(jnp.bfloat16)
    rid = row0 + lax.broadcasted_iota(jnp.int32, (M, 1), 0)

    def body(e, carry):
        s = offs_ref[e]
        t = offs_ref[e + 1]

        @pl.when((s < row0 + M) & (t > row0))
        def _():
            h = jnp.maximum(
                jnp.dot(xo, W1_ref[e], preferred_element_type=jnp.float32)
                + jnp.dot(xe, W1e_ref[e], preferred_element_type=jnp.float32)
                + b1_ref[e], 0.0)
            lg = (jnp.dot(h.astype(jnp.bfloat16), W2_ref[e],
                          preferred_element_type=jnp.float32)
                  + b2_ref[e])
            msk = (rid >= s) & (rid < t)
            lo_ref[...] = jnp.where(msk, lg, lo_ref[...])
        return carry

    lax.fori_loop(0, E, body, 0)


def _moe(offs, xo_s, xe_s, W1, W1e, b1, W2, b2):
    grid_spec = pltpu.PrefetchScalarGridSpec(
        num_scalar_prefetch=1,
        grid=(NB,),
        in_specs=[
            pl.BlockSpec((M, RAWOB), lambda i, s: (i, 0)),
            pl.BlockSpec((M, DPAD), lambda i, s: (i, 0)),
            pl.BlockSpec((E, DH, DH), lambda i, s: (0, 0, 0)),
            pl.BlockSpec((E, DEXT, DH), lambda i, s: (0, 0, 0)),
            pl.BlockSpec((E, DH), lambda i, s: (0, 0)),
            pl.BlockSpec((E, DH, NA), lambda i, s: (0, 0, 0)),
            pl.BlockSpec((E, NA), lambda i, s: (0, 0)),
        ],
        out_specs=[
            pl.BlockSpec((M, NA), lambda i, s: (i, 0)),
        ],
    )
    return pl.pallas_call(
        _moe_body,
        grid_spec=grid_spec,
        out_shape=(
            jax.ShapeDtypeStruct((N, NA), jnp.float32),
        ),
        compiler_params=pltpu.CompilerParams(
            dimension_semantics=("arbitrary",),
            vmem_limit_bytes=100 * 1024 * 1024,
        ),
    )(offs, xo_s, xe_s, W1, W1e, b1, W2, b2)[0]


def _critic_body(obs_ref, ht_ref, gp_ref, Wc1_ref, Wc1e_ref, bc1_ref,
                 Wc2_ref, bc2_ref, vo_ref):
    ht = ht_ref[...]
    gp16 = gp_ref[...]
    lanec = lax.broadcasted_iota(jnp.int32, (M, DEXT), 1)
    htb = jnp.broadcast_to(ht, (M, DEXT))
    gp_rep = jnp.repeat(gp16, A, axis=0)
    gp_pad = jnp.pad(gp_rep, ((0, 0), (1, DEXT - 1 - N_TP)))
    xe = jnp.where(lanec == 0, htb.astype(jnp.float32),
                   jnp.where((lanec >= 1) & (lanec <= N_TP),
                             jnp.where(lanec - 1 == htb, -1.0,
                                       gp_pad.astype(jnp.float32)),
                             0.0))
    xo = obs_ref[...].astype(jnp.bfloat16)
    hc = jnp.maximum(
        jnp.dot(xo, Wc1_ref[...], preferred_element_type=jnp.float32)
        + jnp.dot(xe.astype(jnp.bfloat16), Wc1e_ref[...],
                  preferred_element_type=jnp.float32)
        + bc1_ref[...], 0.0)
    vo_ref[...] = (jnp.dot(hc.astype(jnp.bfloat16), Wc2_ref[...],
                           preferred_element_type=jnp.float32)
                   + bc2_ref[...])


def _critic(obs2, ht2, gp, Wc1, Wc1e, bc1, Wc2p, bc2p):
    return pl.pallas_call(
        _critic_body,
        grid=(NB,),
        in_specs=[
            pl.BlockSpec((M, RAWOB), lambda i: (i, 0)),
            pl.BlockSpec((M, 1), lambda i: (i, 0)),
            pl.BlockSpec((M // A, N_TP), lambda i: (i, 0)),
            pl.BlockSpec((DH, DH), lambda i: (0, 0)),
            pl.BlockSpec((DEXT, DH), lambda i: (0, 0)),
            pl.BlockSpec((1, DH), lambda i: (0, 0)),
            pl.BlockSpec((DH, VW), lambda i: (0, 0)),
            pl.BlockSpec((1, VW), lambda i: (0, 0)),
        ],
        out_specs=pl.BlockSpec((M, VW), lambda i: (i, 0)),
        out_shape=jax.ShapeDtypeStruct((N, VW), jnp.float32),
        compiler_params=pltpu.CompilerParams(
            dimension_semantics=("parallel",),
            vmem_limit_bytes=100 * 1024 * 1024,
        ),
    )(obs2, ht2, gp, Wc1, Wc1e, bc1, Wc2p, bc2p)


def kernel(hete_pick, obs, hete_type, gp_sel_summary, thread_index,
           W1, b1, W2, b2, Wc1, bc1, Wc2, bc2):
    pick = hete_pick.reshape(N).astype(jnp.int32)
    ht = hete_type.reshape(N).astype(jnp.int32)
    gpf = gp_sel_summary.reshape(T * N_TP).astype(jnp.int32)
    obs2 = obs.reshape(N, RAWOB)
    W1e = jnp.pad(W1[:, RAWOB:, :], ((0, 0), (0, DEXT - 5), (0, 0)))
    Wc1e = jnp.pad(Wc1[RAWOB:, :], ((0, DEXT - 5), (0, 0)))
    Wc2p = jnp.pad(Wc2, ((0, 0), (0, VW - 1)))
    W1b = W1.astype(jnp.bfloat16)
    W1eb = W1e.astype(jnp.bfloat16)
    W2b = W2.astype(jnp.bfloat16)
    Wc1b = Wc1.astype(jnp.bfloat16)
    Wc1eb = Wc1e.astype(jnp.bfloat16)
    Wc2pb = Wc2p.astype(jnp.bfloat16)
    bc2p = jnp.pad(bc2, (0, VW - 1)).reshape(1, VW)
    bc1_2 = bc1.reshape(1, DH)

    vs = _critic(obs2, ht.reshape(N, 1), gp_sel_summary.astype(jnp.int32),
                 Wc1b, Wc1eb, bc1_2, Wc2pb, bc2p)
    xo_s, xe_s, dest, offs = _route(pick, ht, gpf, obs2)
    ls = _moe(offs, xo_s, xe_s, W1b, W1eb, b1, W2b, b2)
    lf = _gather(ls, dest)
    return lf.reshape(T, A, NA), vs[:, :1].reshape(T, A, 1)
